# baseline (speedup 1.0000x reference)
import functools

import jax
import jax.numpy as jnp
from jax import lax
from jax.experimental import pallas as pl
from jax.experimental.pallas import tpu as pltpu
from jax.experimental.pallas import tpu_sc as plsc

_NUM_CLASSES = 1000000
_DROPOUT_PROB = 0.1
_LANES = 16
_SLAB = 512
_RING = 32
_CPAD = 536


@functools.lru_cache(maxsize=None)
def _build_sc_gather(batch: int, hidden: int, classes: int):
    info = plsc.get_sparse_core_info()
    nc, ns = info.num_cores, info.num_subcores
    nw = nc * ns
    b_per_w = batch // nw

    mesh = plsc.VectorSubcoreMesh(core_axis_name="c", subcore_axis_name="s")

    @functools.partial(
        pl.kernel,
        out_type=jax.ShapeDtypeStruct((batch, hidden), jnp.float32),
        mesh=mesh,
        compiler_params=pltpu.CompilerParams(needs_layout_passes=False),
        scratch_types=[
            pltpu.VMEM((b_per_w + _LANES,), jnp.int32),
            pltpu.VMEM((b_per_w + _LANES,), jnp.int32),
            pltpu.VMEM((1, _CPAD), jnp.int32),
            pltpu.VMEM((1, _CPAD), jnp.int32),
            pltpu.VMEM((1, _LANES), jnp.int32),
            pltpu.VMEM((2, hidden, _SLAB), jnp.float32),
            pltpu.VMEM((_RING, hidden), jnp.float32),
            pltpu.SemaphoreType.DMA,
            pltpu.SemaphoreType.DMA,
        ],
    )
    def k(lab_hbm, pos_hbm, cs_hbm, cb_hbm, ncnt_hbm, tab_t_hbm, out_hbm,
          lab_v, pos_v, cs_v, cb_v, nc_v, slab_v, ring_v, fsem, wsem):
        wid = lax.axis_index("s") * nc + lax.axis_index("c")
        base = wid * b_per_w
        pltpu.sync_copy(lab_hbm.at[pl.ds(base, b_per_w + _LANES)], lab_v)
        pltpu.sync_copy(pos_hbm.at[pl.ds(base, b_per_w + _LANES)], pos_v)
        pltpu.sync_copy(cs_hbm.at[pl.ds(wid, 1)], cs_v)
        pltpu.sync_copy(cb_hbm.at[pl.ds(wid, 1)], cb_v)
        pltpu.sync_copy(ncnt_hbm.at[pl.ds(wid, 1)], nc_v)

        row_ids = [lax.iota(jnp.int32, 16) + 16 * t for t in range(hidden // 16)]

        def ext(ref, p):
            return ref[pl.ds(p, _LANES)][0]

        def ext2(ref2, p):
            return ref2[0, pl.ds(p, _LANES)][0]

        nch = nc_v[0, pl.ds(0, _LANES)][0]

        def fire(g, par):
            s = ext2(cs_v, g)
            s = pl.multiple_of(s, 128)
            pltpu.async_copy(tab_t_hbm.at[:, pl.ds(s, _SLAB)],
                             slab_v.at[par], fsem)

        fire(0, 0)

        def chunk(g, _):
            par = lax.rem(g, 2)
            pltpu.make_async_copy(
                tab_t_hbm.at[:, pl.ds(0, _SLAB)], slab_v.at[par], fsem).wait()

            @pl.when(g + 1 < nch)
            def _():
                fire(g + 1, lax.rem(g + 1, 2))

            start = ext2(cs_v, g)
            pb = ext2(cb_v, g)
            pe = ext2(cb_v, g + 1)
            pvec = jnp.full((_LANES,), par, jnp.int32)

            def label(p, _):
                lab = ext(lab_v, p)
                col = jnp.full((_LANES,), lab - start, jnp.int32)
                slot = lax.rem(p, _RING)
                for t in range(hidden // 16):
                    vec = plsc.load_gather(slab_v, [pvec, row_ids[t], col])
                    ring_v[slot, pl.ds(t * 16, 16)] = vec
                pos = ext(pos_v, p)
                pltpu.async_copy(ring_v.at[pl.ds(slot, 1)],
                                 out_hbm.at[pl.ds(pos, 1)], wsem)

                @pl.when(slot == _RING - 1)
                def _():
                    pltpu.make_async_copy(
                        ring_v, out_hbm.at[pl.ds(0, _RING)], wsem).wait()

                return 0

            lax.fori_loop(pb, pe, label, 0)
            return 0

        lax.fori_loop(0, nch, chunk, 0)

    return k


def kernel(labels, table, train):
    batch = labels.shape[0]
    classes, hidden = table.shape
    nw = 32
    b_per_w = batch // nw
    classes_pad = -(-classes // 128) * 128
    max_start = classes_pad - _SLAB

    drop_ids = jax.random.uniform(jax.random.key(42), (batch,)) < _DROPOUT_PROB
    gate = jnp.asarray(train) != 0
    masked = jnp.where(drop_ids & gate, classes - 1, labels.astype(jnp.int32))
    pos = lax.iota(jnp.int32, batch)
    sorted_lab, sorted_pos = lax.sort_key_val(masked, pos)

    i = lax.iota(jnp.int32, batch)
    w = i // b_per_w
    p_local = i % b_per_w
    sbin = sorted_lab >> 9
    is_new = jnp.concatenate(
        [jnp.ones((1,), bool), sbin[1:] != sbin[:-1]]) | (p_local == 0)
    seg = (jnp.cumsum(is_new.astype(jnp.int32)) - 1)
    seg = seg - jnp.take(seg, w * b_per_w)
    start_lane = jnp.minimum(sbin << 9, max_start)
    flat = w * _CPAD + seg
    neg = jnp.full((batch,), -1, jnp.int32)
    cs = jnp.full((nw * _CPAD,), -1, jnp.int32).at[flat].max(
        jnp.where(is_new, start_lane, neg))
    cb = jnp.full((nw * _CPAD,), -1, jnp.int32).at[flat].max(
        jnp.where(is_new, p_local, neg))
    cs = jnp.where(cs < 0, 0, cs).reshape(nw, _CPAD)
    cb = jnp.where(cb < 0, b_per_w, cb).reshape(nw, _CPAD)
    ncnt = jnp.zeros((nw * _LANES,), jnp.int32).at[w * _LANES].max(
        seg + 1).reshape(nw, _LANES)

    pad_i = jnp.full((_LANES,), jnp.int32(2**30))
    sorted_lab = jnp.concatenate([sorted_lab, pad_i])
    sorted_pos = jnp.concatenate([sorted_pos, jnp.zeros((_LANES,), jnp.int32)])

    return _build_sc_gather(batch, hidden, classes)(
        sorted_lab, sorted_pos, cs, cb, ncnt, table.T)


# speedup vs baseline: 1.8805x; 1.8805x over previous
import functools

import jax
import jax.numpy as jnp
from jax import lax
from jax.experimental import pallas as pl
from jax.experimental.pallas import tpu as pltpu
from jax.experimental.pallas import tpu_sc as plsc

_NUM_CLASSES = 1000000
_DROPOUT_PROB = 0.1
_LANES = 16
_SLAB = 512
_RING = 32
_OFF = 8
_CPAD = 536


@functools.lru_cache(maxsize=None)
def _build_sc_gather(batch: int, hidden: int, classes: int):
    info = plsc.get_sparse_core_info()
    nc, ns = info.num_cores, info.num_subcores
    nw = nc * ns
    b_per_w = batch // nw
    classes_pad = -(-classes // 128) * 128
    max_start = classes_pad - _SLAB

    mesh = plsc.VectorSubcoreMesh(core_axis_name="c", subcore_axis_name="s")

    @functools.partial(
        pl.kernel,
        out_type=jax.ShapeDtypeStruct((batch, hidden), jnp.float32),
        mesh=mesh,
        compiler_params=pltpu.CompilerParams(needs_layout_passes=False),
        scratch_types=[
            pltpu.VMEM((_OFF + b_per_w + _LANES,), jnp.int32),
            pltpu.VMEM((b_per_w + _LANES,), jnp.int32),
            pltpu.VMEM((_CPAD,), jnp.int32),
            pltpu.VMEM((_CPAD,), jnp.int32),
            pltpu.VMEM((2, hidden, _SLAB), jnp.float32),
            pltpu.VMEM((_RING, hidden), jnp.float32),
            pltpu.SemaphoreType.DMA,
            pltpu.SemaphoreType.DMA,
        ],
    )
    def k(lab_hbm, pos_hbm, tab_t_hbm, out_hbm,
          lab_v, pos_v, cs_v, cb_v, slab_v, ring_v, fsem, wsem):
        wid = lax.axis_index("s") * nc + lax.axis_index("c")
        base = wid * b_per_w
        lab_v[pl.ds(0, _LANES)] = jnp.full((_LANES,), -1, jnp.int32)
        pltpu.sync_copy(lab_hbm.at[pl.ds(base, b_per_w + _LANES)],
                        lab_v.at[pl.ds(_OFF, b_per_w + _LANES)])
        pltpu.sync_copy(pos_hbm.at[pl.ds(base, b_per_w + _LANES)], pos_v)

        row_ids = [lax.iota(jnp.int32, 16) + 16 * t for t in range(hidden // 16)]
        iota16 = lax.iota(jnp.int32, 16)

        def ext(ref, p):
            return ref[pl.ds(p, _LANES)][0]

        def derive(g, cnt):
            labs = lab_v[pl.ds(_OFF + g * 16, 16)]
            prev = lab_v[pl.ds(_OFF + g * 16 - 1, 16)]
            bins = labs >> 9
            is_new = bins != (prev >> 9)
            starts = jnp.minimum(bins << 9, jnp.full((16,), max_start,
                                                     jnp.int32))
            plsc.store_compressed(cs_v.at[pl.ds(cnt, 16)], starts, mask=is_new)
            plsc.store_compressed(cb_v.at[pl.ds(cnt, 16)],
                                  iota16 + g * 16, mask=is_new)
            return cnt + plsc.all_reduce_population_count(is_new)[0]

        nch = lax.fori_loop(0, b_per_w // 16, derive, jnp.int32(0))
        cb_v[pl.ds(nch, 16)] = jnp.full((16,), b_per_w, jnp.int32)

        def fire(g, par):
            s = ext(cs_v, g)
            s = pl.multiple_of(s, 128)
            pltpu.async_copy(tab_t_hbm.at[:, pl.ds(s, _SLAB)],
                             slab_v.at[par], fsem)

        fire(0, 0)

        def chunk(g, _):
            par = lax.rem(g, 2)
            pltpu.make_async_copy(
                tab_t_hbm.at[:, pl.ds(0, _SLAB)], slab_v.at[par], fsem).wait()

            @pl.when(g + 1 < nch)
            def _():
                fire(g + 1, lax.rem(g + 1, 2))

            start = ext(cs_v, g)
            pb = ext(cb_v, g)
            pe = ext(cb_v, g + 1)
            pvec = jnp.full((_LANES,), par, jnp.int32)

            def label(p, _):
                lab = ext(lab_v, _OFF + p)
                col = jnp.full((_LANES,), lab - start, jnp.int32)
                slot = lax.rem(p, _RING)
                for t in range(hidden // 16):
                    vec = plsc.load_gather(slab_v, [pvec, row_ids[t], col])
                    ring_v[slot, pl.ds(t * 16, 16)] = vec
                pos = ext(pos_v, p)
                pltpu.async_copy(ring_v.at[pl.ds(slot, 1)],
                                 out_hbm.at[pl.ds(pos, 1)], wsem)

                @pl.when(slot == _RING - 1)
                def _():
                    pltpu.make_async_copy(
                        ring_v, out_hbm.at[pl.ds(0, _RING)], wsem).wait()

                return 0

            lax.fori_loop(pb, pe, label, 0)
            return 0

        lax.fori_loop(0, nch, chunk, 0)

    return k


def kernel(labels, table, train):
    batch = labels.shape[0]
    classes, hidden = table.shape
    drop_ids = jax.random.uniform(jax.random.key(42), (batch,)) < _DROPOUT_PROB
    gate = jnp.asarray(train) != 0
    masked = jnp.where(drop_ids & gate, classes - 1, labels.astype(jnp.int32))
    pos = lax.iota(jnp.int32, batch)
    sorted_lab, sorted_pos = lax.sort_key_val(masked, pos)
    pad_i = jnp.full((_LANES,), jnp.int32(2**30))
    sorted_lab = jnp.concatenate([sorted_lab, pad_i])
    sorted_pos = jnp.concatenate([sorted_pos, jnp.zeros((_LANES,), jnp.int32)])
    return _build_sc_gather(batch, hidden, classes)(
        sorted_lab, sorted_pos, table.T)


# speedup vs baseline: 2.3207x; 1.2341x over previous
import functools

import jax
import jax.numpy as jnp
from jax import lax
from jax.experimental import pallas as pl
from jax.experimental.pallas import tpu as pltpu
from jax.experimental.pallas import tpu_sc as plsc

_NUM_CLASSES = 1000000
_DROPOUT_PROB = 0.1
_LANES = 16
_SLAB = 512
_RING = 32
_OFF = 8
_CPAD = 536


@functools.lru_cache(maxsize=None)
def _build_sc_gather(batch: int, hidden: int, classes: int):
    info = plsc.get_sparse_core_info()
    nc, ns = info.num_cores, info.num_subcores
    nw = nc * ns
    b_per_w = batch // nw
    classes_pad = -(-classes // 128) * 128
    max_start = classes_pad - _SLAB

    mesh = plsc.VectorSubcoreMesh(core_axis_name="c", subcore_axis_name="s")

    @functools.partial(
        pl.kernel,
        out_type=jax.ShapeDtypeStruct((batch, hidden), jnp.float32),
        mesh=mesh,
        compiler_params=pltpu.CompilerParams(needs_layout_passes=False),
        scratch_types=[
            pltpu.VMEM((_OFF + b_per_w + _LANES,), jnp.int32),
            pltpu.VMEM((b_per_w + _LANES,), jnp.int32),
            pltpu.VMEM((_CPAD,), jnp.int32),
            pltpu.VMEM((_CPAD,), jnp.int32),
            pltpu.VMEM((3, hidden, _SLAB), jnp.float32),
            pltpu.VMEM((_RING, hidden), jnp.float32),
            pltpu.SemaphoreType.DMA,
            pltpu.SemaphoreType.DMA,
            pltpu.SemaphoreType.DMA,
            pltpu.SemaphoreType.DMA,
        ],
    )
    def k(lab_hbm, pos_hbm, tab_t_hbm, out_hbm,
          lab_v, pos_v, cs_v, cb_v, slab_v, ring_v, f0, f1, f2, wsem):
        fsems = (f0, f1, f2)
        wid = lax.axis_index("s") * nc + lax.axis_index("c")
        base = wid * b_per_w
        lab_v[pl.ds(0, _LANES)] = jnp.full((_LANES,), -1, jnp.int32)
        pltpu.sync_copy(lab_hbm.at[pl.ds(base, b_per_w + _LANES)],
                        lab_v.at[pl.ds(_OFF, b_per_w + _LANES)])
        pltpu.sync_copy(pos_hbm.at[pl.ds(base, b_per_w + _LANES)], pos_v)

        row_ids = [lax.iota(jnp.int32, 16) + 16 * t for t in range(hidden // 16)]
        iota16 = lax.iota(jnp.int32, 16)

        def ext(ref, p):
            return ref[pl.ds(p, _LANES)][0]

        def derive(g, cnt):
            labs = lab_v[pl.ds(_OFF + g * 16, 16)]
            prev = lab_v[pl.ds(_OFF + g * 16 - 1, 16)]
            bins = labs >> 9
            is_new = bins != (prev >> 9)
            starts = jnp.minimum(bins << 9, jnp.full((16,), max_start,
                                                     jnp.int32))
            plsc.store_compressed(cs_v.at[pl.ds(cnt, 16)], starts, mask=is_new)
            plsc.store_compressed(cb_v.at[pl.ds(cnt, 16)],
                                  iota16 + g * 16, mask=is_new)
            return cnt + plsc.all_reduce_population_count(is_new)[0]

        nch = lax.fori_loop(0, b_per_w // 16, derive, jnp.int32(0))
        cb_v[pl.ds(nch, 16)] = jnp.full((16,), b_per_w, jnp.int32)

        def fire(g):
            par = lax.rem(g, 3)
            s = ext(cs_v, g)
            s = pl.multiple_of(s, 128)
            for i in range(3):
                @pl.when(par == i)
                def _(i=i):
                    pltpu.async_copy(tab_t_hbm.at[:, pl.ds(s, _SLAB)],
                                     slab_v.at[i], fsems[i])

        fire(0)

        @pl.when(nch > 1)
        def _():
            fire(1)

        def chunk(g, _):
            par = lax.rem(g, 3)
            for i in range(3):
                @pl.when(par == i)
                def _(i=i):
                    pltpu.make_async_copy(
                        tab_t_hbm.at[:, pl.ds(0, _SLAB)], slab_v.at[i],
                        fsems[i]).wait()

            @pl.when(g + 2 < nch)
            def _():
                fire(g + 2)

            start = ext(cs_v, g)
            pb = ext(cb_v, g)
            pe = ext(cb_v, g + 1)
            pvec = jnp.full((_LANES,), par, jnp.int32)

            def label(p, _):
                lab = ext(lab_v, _OFF + p)
                col = jnp.full((_LANES,), lab - start, jnp.int32)
                slot = lax.rem(p, _RING)
                for t in range(hidden // 16):
                    vec = plsc.load_gather(slab_v, [pvec, row_ids[t], col])
                    ring_v[slot, pl.ds(t * 16, 16)] = vec
                pos = ext(pos_v, p)
                pltpu.async_copy(ring_v.at[pl.ds(slot, 1)],
                                 out_hbm.at[pl.ds(pos, 1)], wsem)

                @pl.when(slot == _RING - 1)
                def _():
                    pltpu.make_async_copy(
                        ring_v, out_hbm.at[pl.ds(0, _RING)], wsem).wait()

                return 0

            lax.fori_loop(pb, pe, label, 0)
            return 0

        lax.fori_loop(0, nch, chunk, 0)

    return k


def kernel(labels, table, train):
    batch = labels.shape[0]
    classes, hidden = table.shape
    drop_ids = jax.random.uniform(jax.random.key(42), (batch,)) < _DROPOUT_PROB
    gate = jnp.asarray(train) != 0
    masked = jnp.where(drop_ids & gate, classes - 1, labels.astype(jnp.int32))
    pos = lax.iota(jnp.int32, batch)
    sorted_lab, sorted_pos = lax.sort_key_val(masked, pos)
    pad_i = jnp.full((_LANES,), jnp.int32(2**30))
    sorted_lab = jnp.concatenate([sorted_lab, pad_i])
    sorted_pos = jnp.concatenate([sorted_pos, jnp.zeros((_LANES,), jnp.int32)])
    return _build_sc_gather(batch, hidden, classes)(
        sorted_lab, sorted_pos, table.T)


# speedup vs baseline: 2.3307x; 1.0043x over previous
import functools

import jax
import jax.numpy as jnp
from jax import lax
from jax.experimental import pallas as pl
from jax.experimental.pallas import tpu as pltpu
from jax.experimental.pallas import tpu_sc as plsc

_NUM_CLASSES = 1000000
_DROPOUT_PROB = 0.1
_LANES = 16
_SLAB = 512
_RING = 64
_OFF = 8
_CPAD = 536


@functools.lru_cache(maxsize=None)
def _build_sc_gather(batch: int, hidden: int, classes: int):
    info = plsc.get_sparse_core_info()
    nc, ns = info.num_cores, info.num_subcores
    nw = nc * ns
    b_per_w = batch // nw
    classes_pad = -(-classes // 128) * 128
    max_start = classes_pad - _SLAB

    mesh = plsc.VectorSubcoreMesh(core_axis_name="c", subcore_axis_name="s")

    @functools.partial(
        pl.kernel,
        out_type=jax.ShapeDtypeStruct((batch, hidden), jnp.float32),
        mesh=mesh,
        compiler_params=pltpu.CompilerParams(needs_layout_passes=False),
        scratch_types=[
            pltpu.VMEM((_OFF + b_per_w + _LANES,), jnp.int32),
            pltpu.VMEM((b_per_w + _LANES,), jnp.int32),
            pltpu.VMEM((_CPAD,), jnp.int32),
            pltpu.VMEM((_CPAD,), jnp.int32),
            pltpu.VMEM((3, hidden, _SLAB), jnp.float32),
            pltpu.VMEM((_RING, hidden), jnp.float32),
            pltpu.SemaphoreType.DMA,
            pltpu.SemaphoreType.DMA,
            pltpu.SemaphoreType.DMA,
            pltpu.SemaphoreType.DMA,
        ],
    )
    def k(lab_hbm, pos_hbm, tab_t_hbm, out_hbm,
          lab_v, pos_v, cs_v, cb_v, slab_v, ring_v, f0, f1, f2, wsem):
        fsems = (f0, f1, f2)
        wid = lax.axis_index("s") * nc + lax.axis_index("c")
        base = wid * b_per_w
        lab_v[pl.ds(0, _LANES)] = jnp.full((_LANES,), -1, jnp.int32)
        pltpu.sync_copy(lab_hbm.at[pl.ds(base, b_per_w + _LANES)],
                        lab_v.at[pl.ds(_OFF, b_per_w + _LANES)])
        pltpu.sync_copy(pos_hbm.at[pl.ds(base, b_per_w + _LANES)], pos_v)

        row_ids = [lax.iota(jnp.int32, 16) + 16 * t for t in range(hidden // 16)]
        iota16 = lax.iota(jnp.int32, 16)

        def ext(ref, p):
            return ref[pl.ds(p, _LANES)][0]

        def derive(g, cnt):
            labs = lab_v[pl.ds(_OFF + g * 16, 16)]
            prev = lab_v[pl.ds(_OFF + g * 16 - 1, 16)]
            bins = labs >> 9
            is_new = bins != (prev >> 9)
            starts = jnp.minimum(bins << 9, jnp.full((16,), max_start,
                                                     jnp.int32))
            plsc.store_compressed(cs_v.at[pl.ds(cnt, 16)], starts, mask=is_new)
            plsc.store_compressed(cb_v.at[pl.ds(cnt, 16)],
                                  iota16 + g * 16, mask=is_new)
            return cnt + plsc.all_reduce_population_count(is_new)[0]

        nch = lax.fori_loop(0, b_per_w // 16, derive, jnp.int32(0))
        cb_v[pl.ds(nch, 16)] = jnp.full((16,), b_per_w, jnp.int32)

        def fire(g):
            par = lax.rem(g, 3)
            s = ext(cs_v, g)
            s = pl.multiple_of(s, 128)
            for i in range(3):
                @pl.when(par == i)
                def _(i=i):
                    pltpu.async_copy(tab_t_hbm.at[:, pl.ds(s, _SLAB)],
                                     slab_v.at[i], fsems[i])

        fire(0)

        @pl.when(nch > 1)
        def _():
            fire(1)

        def chunk(g, _):
            par = lax.rem(g, 3)
            for i in range(3):
                @pl.when(par == i)
                def _(i=i):
                    pltpu.make_async_copy(
                        tab_t_hbm.at[:, pl.ds(0, _SLAB)], slab_v.at[i],
                        fsems[i]).wait()

            @pl.when(g + 2 < nch)
            def _():
                fire(g + 2)

            start = ext(cs_v, g)
            pb = ext(cb_v, g)
            pe = ext(cb_v, g + 1)
            pvec = jnp.full((_LANES,), par, jnp.int32)

            def label(p, _):
                lab = ext(lab_v, _OFF + p)
                col = jnp.full((_LANES,), lab - start, jnp.int32)
                slot = lax.rem(p, _RING)
                for t in range(hidden // 16):
                    vec = plsc.load_gather(slab_v, [pvec, row_ids[t], col])
                    ring_v[slot, pl.ds(t * 16, 16)] = vec
                pos = ext(pos_v, p)
                pltpu.async_copy(ring_v.at[pl.ds(slot, 1)],
                                 out_hbm.at[pl.ds(pos, 1)], wsem)

                @pl.when(slot == _RING - 1)
                def _():
                    pltpu.make_async_copy(
                        ring_v, out_hbm.at[pl.ds(0, _RING)], wsem).wait()

                return 0

            lax.fori_loop(pb, pe, label, 0)
            return 0

        lax.fori_loop(0, nch, chunk, 0)

    return k


def kernel(labels, table, train):
    batch = labels.shape[0]
    classes, hidden = table.shape
    drop_ids = jax.random.uniform(jax.random.key(42), (batch,)) < _DROPOUT_PROB
    gate = jnp.asarray(train) != 0
    masked = jnp.where(drop_ids & gate, classes - 1, labels.astype(jnp.int32))
    pos = lax.iota(jnp.int32, batch)
    sorted_lab, sorted_pos = lax.sort_key_val(masked, pos)
    pad_i = jnp.full((_LANES,), jnp.int32(2**30))
    sorted_lab = jnp.concatenate([sorted_lab, pad_i])
    sorted_pos = jnp.concatenate([sorted_pos, jnp.zeros((_LANES,), jnp.int32)])
    return _build_sc_gather(batch, hidden, classes)(
        sorted_lab, sorted_pos, table.T)
